# baseline (speedup 1.0000x reference)
import functools

import jax
import jax.numpy as jnp
from jax import lax
from jax.experimental import pallas as pl
from jax.experimental.pallas import tpu as pltpu
from jax.experimental.pallas import tpu_sc as plsc

N_FIELDS = 26
VOCAB = 100000
EMBED_DIM = 16
BATCH = 16384

NC = 2
NS = 16
NW = NC * NS

TOTAL_ROWS = BATCH * N_FIELDS
ROWS_PER_W = TOTAL_ROWS // NW
GWIN = 128
NWIN = ROWS_PER_W // GWIN
GROUP = 13
GCHUNK = GROUP * GWIN
NGROUPS = NWIN // GROUP

BLK = 512


def _sc_gather(flat_tables, gidx):
    mesh = plsc.VectorSubcoreMesh(core_axis_name="c", subcore_axis_name="s")

    @functools.partial(
        pl.kernel,
        mesh=mesh,
        compiler_params=pltpu.CompilerParams(use_tc_tiling_on_sc=False),
        out_type=jax.ShapeDtypeStruct((TOTAL_ROWS, EMBED_DIM), jnp.float32),
        scratch_types=[
            pltpu.VMEM((NWIN, GWIN), jnp.int32),
            pltpu.VMEM((GCHUNK, EMBED_DIM), jnp.float32),
            pltpu.VMEM((GCHUNK, EMBED_DIM), jnp.float32),
            pltpu.SemaphoreType.DMA,
            pltpu.SemaphoreType.DMA,
            pltpu.SemaphoreType.DMA,
            pltpu.SemaphoreType.DMA,
        ],
    )
    def k(tbl_hbm, idx_hbm, out_hbm, idx_v, rows_a, rows_b, gsem_a, gsem_b,
          wsem_a, wsem_b):
        wid = lax.axis_index("s") * NC + lax.axis_index("c")
        pltpu.sync_copy(idx_hbm.at[wid], idx_v)
        bufs = ((rows_a, gsem_a, wsem_a), (rows_b, gsem_b, wsem_b))

        @pl.loop(0, NGROUPS, step=2)
        def _(go):
            for s in range(2):
                buf, gsem, wsem = bufs[s]
                g = go + s
                row0 = wid * ROWS_PER_W + g * GCHUNK

                @pl.when(go > 0)
                def _():
                    pltpu.make_async_copy(
                        buf, out_hbm.at[pl.ds(0, GCHUNK)], wsem).wait()

                cps = [
                    pltpu.make_async_copy(
                        tbl_hbm.at[idx_v.at[g * GROUP + t]],
                        buf.at[pl.ds(t * GWIN, GWIN)],
                        gsem)
                    for t in range(GROUP)
                ]
                for cp in cps:
                    cp.start()
                for cp in cps:
                    cp.wait()
                pltpu.make_async_copy(
                    buf, out_hbm.at[pl.ds(row0, GCHUNK)], wsem).start()

        for buf, _, wsem in bufs:
            pltpu.make_async_copy(
                buf, out_hbm.at[pl.ds(0, GCHUNK)], wsem).wait()

    return k(flat_tables, gidx)


def _tc_body(st_ref, w_ref, b_ref, o_ref):
    e = st_ref[...]
    s = jnp.sum(e, axis=0)
    q = jnp.sum(e * e, axis=(0, 2))
    lin = jnp.sum(e * w_ref[...][:, None, :], axis=(0, 2))
    y = lin + b_ref[0] + 0.5 * (jnp.sum(s * s, axis=1) - q)
    o_ref[...] = jax.nn.sigmoid(y)


def _tc_reduce(staged3, w2, b):
    return pl.pallas_call(
        _tc_body,
        grid=(BATCH // BLK,),
        in_specs=[
            pl.BlockSpec((N_FIELDS, BLK, EMBED_DIM), lambda i: (0, i, 0)),
            pl.BlockSpec((N_FIELDS, EMBED_DIM), lambda i: (0, 0)),
            pl.BlockSpec(memory_space=pltpu.SMEM),
        ],
        out_specs=pl.BlockSpec((BLK,), lambda i: (i,)),
        out_shape=jax.ShapeDtypeStruct((BATCH,), jnp.float32),
    )(staged3, w2, b)


def kernel(x, tables, W, b):
    flat_tables = tables.reshape(N_FIELDS * VOCAB, EMBED_DIM)
    offs = (jnp.arange(N_FIELDS, dtype=jnp.int32) * VOCAB)[:, None]
    gidx = (x.T.astype(jnp.int32) + offs).reshape(NW, NWIN, GWIN)
    staged = _sc_gather(flat_tables, gidx)
    staged3 = staged.reshape(N_FIELDS, BATCH, EMBED_DIM)
    w2 = W.reshape(N_FIELDS, EMBED_DIM)
    return _tc_reduce(staged3, w2, b)


# speedup vs baseline: 4.7907x; 4.7907x over previous
import functools

import jax
import jax.numpy as jnp
from jax import lax
from jax.experimental import pallas as pl
from jax.experimental.pallas import tpu as pltpu
from jax.experimental.pallas import tpu_sc as plsc

N_FIELDS = 26
VOCAB = 100000
EMBED_DIM = 16
BATCH = 16384

NC = 2
NS = 16
NW = NC * NS

TOTAL_ROWS = BATCH * N_FIELDS
ROWS_PER_W = TOTAL_ROWS // NW
GWIN = 128
NWIN = ROWS_PER_W // GWIN
GROUP = 13
GCHUNK = GROUP * GWIN
NGROUPS = NWIN // GROUP

VB = 2048
VOCAB_PAD = 49 * VB
FGP = 4
BLKL = 256


def _tr_body(t_ref, o_ref):
    o_ref[0] = t_ref[...].T


def _tc_transpose(tables_2d):
    return pl.pallas_call(
        _tr_body,
        grid=(FGP, VOCAB_PAD // VB),
        in_specs=[pl.BlockSpec((128, VB), lambda g, j: (g, j))],
        out_specs=pl.BlockSpec((1, VB, 128), lambda g, j: (g, j, 0)),
        out_shape=jax.ShapeDtypeStruct((FGP, VOCAB_PAD, 128), jnp.float32),
    )(tables_2d)


def _sc_gather(flat_tables, gidx):
    mesh = plsc.VectorSubcoreMesh(core_axis_name="c", subcore_axis_name="s")

    @functools.partial(
        pl.kernel,
        mesh=mesh,
        compiler_params=pltpu.CompilerParams(use_tc_tiling_on_sc=False),
        out_type=jax.ShapeDtypeStruct((TOTAL_ROWS, EMBED_DIM), jnp.float32),
        scratch_types=[
            pltpu.VMEM((NWIN, GWIN), jnp.int32),
            pltpu.VMEM((GCHUNK, EMBED_DIM), jnp.float32),
            pltpu.VMEM((GCHUNK, EMBED_DIM), jnp.float32),
            pltpu.SemaphoreType.DMA,
            pltpu.SemaphoreType.DMA,
            pltpu.SemaphoreType.DMA,
            pltpu.SemaphoreType.DMA,
        ],
    )
    def k(tbl_hbm, idx_hbm, out_hbm, idx_v, rows_a, rows_b, gsem_a, gsem_b,
          wsem_a, wsem_b):
        wid = lax.axis_index("s") * NC + lax.axis_index("c")
        pltpu.sync_copy(idx_hbm.at[wid], idx_v)
        bufs = ((rows_a, gsem_a, wsem_a), (rows_b, gsem_b, wsem_b))

        @pl.loop(0, NGROUPS, step=2)
        def _(go):
            for s in range(2):
                buf, gsem, wsem = bufs[s]
                g = go + s
                row0 = wid * ROWS_PER_W + g * GCHUNK

                @pl.when(go > 0)
                def _():
                    pltpu.make_async_copy(
                        buf, out_hbm.at[pl.ds(0, GCHUNK)], wsem).wait()

                cps = [
                    pltpu.make_async_copy(
                        tbl_hbm.at[idx_v.at[g * GROUP + t]],
                        buf.at[pl.ds(t * GWIN, GWIN)],
                        gsem)
                    for t in range(GROUP)
                ]
                for cp in cps:
                    cp.start()
                for cp in cps:
                    cp.wait()
                pltpu.make_async_copy(
                    buf, out_hbm.at[pl.ds(row0, GCHUNK)], wsem).start()

        for buf, _, wsem in bufs:
            pltpu.make_async_copy(
                buf, out_hbm.at[pl.ds(0, GCHUNK)], wsem).wait()

    return k(flat_tables, gidx)


def _red_body(st_ref, w_ref, b_ref, o_ref):
    e = st_ref[...]
    s = jnp.sum(e, axis=0)
    q = jnp.sum(e * e, axis=0)
    t = jnp.sum(e * w_ref[...], axis=0)
    z = t + 0.5 * (s * s - q)
    li = lax.broadcasted_iota(jnp.int32, (128, 8), 0) // EMBED_DIM
    ci = lax.broadcasted_iota(jnp.int32, (128, 8), 1)
    g = (li == ci).astype(jnp.float32)
    y8 = lax.dot_general(z, g, (((1,), (0,)), ((), ())),
                         preferred_element_type=jnp.float32)
    o_ref[...] = jax.nn.sigmoid(y8 + b_ref[0])


def _tc_reduce(stg, w128, b):
    nlines = BATCH // 8
    return pl.pallas_call(
        _red_body,
        grid=(nlines // BLKL,),
        in_specs=[
            pl.BlockSpec((N_FIELDS, BLKL, 128), lambda i: (0, i, 0)),
            pl.BlockSpec((N_FIELDS, 1, 128), lambda i: (0, 0, 0)),
            pl.BlockSpec(memory_space=pltpu.SMEM),
        ],
        out_specs=pl.BlockSpec((BLKL, 8), lambda i: (i, 0)),
        out_shape=jax.ShapeDtypeStruct((nlines, 8), jnp.float32),
    )(stg, w128, b)


def kernel(x, tables, W, b):
    tables_t = jnp.swapaxes(tables, 1, 2)
    tables_2d = tables_t.reshape(N_FIELDS * EMBED_DIM, VOCAB)
    tbl_rm = _tc_transpose(tables_2d)
    flat_tables = tbl_rm.reshape(FGP * VOCAB_PAD * 8, EMBED_DIM)
    f = jnp.arange(N_FIELDS, dtype=jnp.int32)[:, None]
    offs = (f // 8) * (VOCAB_PAD * 8) + (f % 8)
    gidx = (x.T.astype(jnp.int32) * 8 + offs).reshape(NW, NWIN, GWIN)
    staged = _sc_gather(flat_tables, gidx)
    stg = staged.reshape(N_FIELDS, BATCH // 8, 128)
    w128 = jnp.tile(W.reshape(N_FIELDS, 1, EMBED_DIM), (1, 1, 8))
    out8 = _tc_reduce(stg, w128, b)
    return out8.reshape(BATCH)


# speedup vs baseline: 5.9846x; 1.2492x over previous
import functools

import jax
import jax.numpy as jnp
from jax import lax
from jax.experimental import pallas as pl
from jax.experimental.pallas import tpu as pltpu
from jax.experimental.pallas import tpu_sc as plsc

N_FIELDS = 26
VOCAB = 100000
EMBED_DIM = 16
BATCH = 16384

NC = 2
NS = 16
NW = NC * NS

TOTAL_ROWS = BATCH * N_FIELDS
ROWS_PER_W = TOTAL_ROWS // NW
GWIN = 128
NWIN = ROWS_PER_W // GWIN
GROUP = 13
GCHUNK = GROUP * GWIN
NGROUPS = NWIN // GROUP

VB = 4096
VOCAB_PAD = 25 * VB
FGP = 4
BLKL = 256


def _tr_body(t_ref, o_ref):
    o_ref[0] = t_ref[...].T


def _tc_transpose(tables_2d):
    return pl.pallas_call(
        _tr_body,
        grid=(FGP, VOCAB_PAD // VB),
        in_specs=[pl.BlockSpec((128, VB), lambda g, j: (g, j))],
        out_specs=pl.BlockSpec((1, VB, 128), lambda g, j: (g, j, 0)),
        out_shape=jax.ShapeDtypeStruct((FGP, VOCAB_PAD, 128), jnp.float32),
    )(tables_2d)


def _sc_gather(flat_tables, gidx):
    mesh = plsc.VectorSubcoreMesh(core_axis_name="c", subcore_axis_name="s")

    @functools.partial(
        pl.kernel,
        mesh=mesh,
        compiler_params=pltpu.CompilerParams(use_tc_tiling_on_sc=False),
        out_type=jax.ShapeDtypeStruct((TOTAL_ROWS, EMBED_DIM), jnp.float32),
        scratch_types=[
            pltpu.VMEM((NWIN, GWIN), jnp.int32),
            pltpu.VMEM((GCHUNK, EMBED_DIM), jnp.float32),
            pltpu.VMEM((GCHUNK, EMBED_DIM), jnp.float32),
            pltpu.SemaphoreType.DMA,
            pltpu.SemaphoreType.DMA,
            pltpu.SemaphoreType.DMA,
            pltpu.SemaphoreType.DMA,
        ],
    )
    def k(tbl_hbm, idx_hbm, out_hbm, idx_v, rows_a, rows_b, gsem_a, gsem_b,
          wsem_a, wsem_b):
        wid = lax.axis_index("s") * NC + lax.axis_index("c")
        pltpu.sync_copy(idx_hbm.at[wid], idx_v)
        bufs = ((rows_a, gsem_a, wsem_a), (rows_b, gsem_b, wsem_b))

        @pl.loop(0, NGROUPS, step=2)
        def _(go):
            for s in range(2):
                buf, gsem, wsem = bufs[s]
                g = go + s
                row0 = wid * ROWS_PER_W + g * GCHUNK

                @pl.when(go > 0)
                def _():
                    pltpu.make_async_copy(
                        buf, out_hbm.at[pl.ds(0, GCHUNK)], wsem).wait()

                cps = [
                    pltpu.make_async_copy(
                        tbl_hbm.at[idx_v.at[g * GROUP + t]],
                        buf.at[pl.ds(t * GWIN, GWIN)],
                        gsem)
                    for t in range(GROUP)
                ]
                for cp in cps:
                    cp.start()
                for cp in cps:
                    cp.wait()
                pltpu.make_async_copy(
                    buf, out_hbm.at[pl.ds(row0, GCHUNK)], wsem).start()

        for buf, _, wsem in bufs:
            pltpu.make_async_copy(
                buf, out_hbm.at[pl.ds(0, GCHUNK)], wsem).wait()

    return k(flat_tables, gidx)


def _red_body(st_ref, w_ref, b_ref, o_ref):
    e = st_ref[...]
    s = jnp.sum(e, axis=0)
    q = jnp.sum(e * e, axis=0)
    t = jnp.sum(e * w_ref[...], axis=0)
    z = t + 0.5 * (s * s - q)
    li = lax.broadcasted_iota(jnp.int32, (128, 8), 0) // EMBED_DIM
    ci = lax.broadcasted_iota(jnp.int32, (128, 8), 1)
    g = (li == ci).astype(jnp.float32)
    y8 = lax.dot_general(z, g, (((1,), (0,)), ((), ())),
                         preferred_element_type=jnp.float32)
    o_ref[...] = jax.nn.sigmoid(y8 + b_ref[0])


def _tc_reduce(stg, w128, b):
    nlines = BATCH // 8
    return pl.pallas_call(
        _red_body,
        grid=(nlines // BLKL,),
        in_specs=[
            pl.BlockSpec((N_FIELDS, BLKL, 128), lambda i: (0, i, 0)),
            pl.BlockSpec((N_FIELDS, 1, 128), lambda i: (0, 0, 0)),
            pl.BlockSpec(memory_space=pltpu.SMEM),
        ],
        out_specs=pl.BlockSpec((BLKL, 8), lambda i: (i, 0)),
        out_shape=jax.ShapeDtypeStruct((nlines, 8), jnp.float32),
    )(stg, w128, b)


def kernel(x, tables, W, b):
    tables_t = jnp.swapaxes(tables, 1, 2)
    tables_2d = tables_t.reshape(N_FIELDS * EMBED_DIM, VOCAB)
    tbl_rm = _tc_transpose(tables_2d)
    flat_tables = tbl_rm.reshape(FGP * VOCAB_PAD * 8, EMBED_DIM)
    f = jnp.arange(N_FIELDS, dtype=jnp.int32)[:, None]
    offs = (f // 8) * (VOCAB_PAD * 8) + (f % 8)
    gidx = (x.T.astype(jnp.int32) * 8 + offs).reshape(NW, NWIN, GWIN)
    staged = _sc_gather(flat_tables, gidx)
    stg = staged.reshape(N_FIELDS, BATCH // 8, 128)
    w128 = jnp.tile(W.reshape(N_FIELDS, 1, EMBED_DIM), (1, 1, 8))
    out8 = _tc_reduce(stg, w128, b)
    return out8.reshape(BATCH)


# speedup vs baseline: 6.2983x; 1.0524x over previous
import functools

import jax
import jax.numpy as jnp
from jax import lax
from jax.experimental import pallas as pl
from jax.experimental.pallas import tpu as pltpu
from jax.experimental.pallas import tpu_sc as plsc

N_FIELDS = 26
VOCAB = 100000
EMBED_DIM = 16
BATCH = 16384

NC = 2
NS = 16
NW = NC * NS

TOTAL_ROWS = BATCH * N_FIELDS
ROWS_PER_W = TOTAL_ROWS // NW
GWIN = 128
NWIN = ROWS_PER_W // GWIN
GROUP = 13
GCHUNK = GROUP * GWIN
NGROUPS = NWIN // GROUP

VB = 4096
VOCAB_PAD = 25 * VB
FGP = 4
NLINES = FGP * (VOCAB_PAD // 2) * 8

BLKL = 256


def _tr_body(t_ref, o_ref):
    t = t_ref[...].T
    o_ref[0] = pltpu.bitcast(t.astype(jnp.bfloat16), jnp.float32)


def _tc_transpose(tables_2d):
    return pl.pallas_call(
        _tr_body,
        grid=(FGP, VOCAB_PAD // VB),
        compiler_params=pltpu.CompilerParams(
            dimension_semantics=("parallel", "parallel")),
        in_specs=[pl.BlockSpec((128, VB), lambda g, j: (g, j))],
        out_specs=pl.BlockSpec((1, VB // 2, 128), lambda g, j: (g, j, 0)),
        out_shape=jax.ShapeDtypeStruct((FGP, VOCAB_PAD // 2, 128),
                                       jnp.float32),
    )(tables_2d)


def _sc_gather(flat_tables, gidx):
    mesh = plsc.VectorSubcoreMesh(core_axis_name="c", subcore_axis_name="s")

    @functools.partial(
        pl.kernel,
        mesh=mesh,
        compiler_params=pltpu.CompilerParams(use_tc_tiling_on_sc=False),
        out_type=jax.ShapeDtypeStruct((TOTAL_ROWS, EMBED_DIM), jnp.float32),
        scratch_types=[
            pltpu.VMEM((NWIN, GWIN), jnp.int32),
            pltpu.VMEM((GCHUNK, EMBED_DIM), jnp.float32),
            pltpu.VMEM((GCHUNK, EMBED_DIM), jnp.float32),
            pltpu.SemaphoreType.DMA,
            pltpu.SemaphoreType.DMA,
            pltpu.SemaphoreType.DMA,
            pltpu.SemaphoreType.DMA,
        ],
    )
    def k(tbl_hbm, idx_hbm, out_hbm, idx_v, rows_a, rows_b, gsem_a, gsem_b,
          wsem_a, wsem_b):
        wid = lax.axis_index("s") * NC + lax.axis_index("c")
        pltpu.sync_copy(idx_hbm.at[wid], idx_v)
        bufs = ((rows_a, gsem_a, wsem_a), (rows_b, gsem_b, wsem_b))

        @pl.loop(0, NGROUPS, step=2)
        def _(go):
            for s in range(2):
                buf, gsem, wsem = bufs[s]
                g = go + s
                row0 = wid * ROWS_PER_W + g * GCHUNK

                @pl.when(go > 0)
                def _():
                    pltpu.make_async_copy(
                        buf, out_hbm.at[pl.ds(0, GCHUNK)], wsem).wait()

                cps = [
                    pltpu.make_async_copy(
                        tbl_hbm.at[idx_v.at[g * GROUP + t]],
                        buf.at[pl.ds(t * GWIN, GWIN)],
                        gsem)
                    for t in range(GROUP)
                ]
                for cp in cps:
                    cp.start()
                for cp in cps:
                    cp.wait()
                pltpu.make_async_copy(
                    buf, out_hbm.at[pl.ds(row0, GCHUNK)], wsem).start()

        for buf, _, wsem in bufs:
            pltpu.make_async_copy(
                buf, out_hbm.at[pl.ds(0, GCHUNK)], wsem).wait()

    return k(flat_tables, gidx)


def _group_mat(transpose=False):
    li = lax.broadcasted_iota(jnp.int32, (128, 8), 0) // EMBED_DIM
    ci = lax.broadcasted_iota(jnp.int32, (128, 8), 1)
    g = (li == ci).astype(jnp.float32)
    return g.T if transpose else g


def _red_body(st_ref, par_ref, w_ref, b_ref, o_ref):
    ew = st_ref[...]
    ui = lax.bitcast_convert_type(ew, jnp.int32)
    lo = lax.bitcast_convert_type(ui << 16, jnp.float32)
    hi = lax.bitcast_convert_type(
        ui & jnp.int32(-65536), jnp.float32)
    par8 = par_ref[...].reshape(N_FIELDS * BLKL, 8).astype(jnp.float32)
    par = lax.dot_general(par8, _group_mat(True), (((1,), (0,)), ((), ())),
                          preferred_element_type=jnp.float32)
    par = par.reshape(N_FIELDS, BLKL, 128)
    e = jnp.where(par > 0.5, hi, lo)
    s = jnp.sum(e, axis=0)
    q = jnp.sum(e * e, axis=0)
    t = jnp.sum(e * w_ref[...], axis=0)
    z = t + 0.5 * (s * s - q)
    y8 = lax.dot_general(z, _group_mat(), (((1,), (0,)), ((), ())),
                         preferred_element_type=jnp.float32)
    o_ref[...] = jax.nn.sigmoid(y8 + b_ref[0])


def _tc_reduce(stg, parr, w128, b):
    nlines = BATCH // 8
    return pl.pallas_call(
        _red_body,
        grid=(nlines // BLKL,),
        compiler_params=pltpu.CompilerParams(
            dimension_semantics=("parallel",)),
        in_specs=[
            pl.BlockSpec((N_FIELDS, BLKL, 128), lambda i: (0, i, 0)),
            pl.BlockSpec((N_FIELDS, BLKL, 8), lambda i: (0, i, 0)),
            pl.BlockSpec((N_FIELDS, 1, 128), lambda i: (0, 0, 0)),
            pl.BlockSpec(memory_space=pltpu.SMEM),
        ],
        out_specs=pl.BlockSpec((BLKL, 8), lambda i: (i, 0)),
        out_shape=jax.ShapeDtypeStruct((nlines, 8), jnp.float32),
    )(stg, parr, w128, b)


def kernel(x, tables, W, b):
    tables_t = jnp.swapaxes(tables, 1, 2)
    tables_2d = tables_t.reshape(N_FIELDS * EMBED_DIM, VOCAB)
    tbl_rm = _tc_transpose(tables_2d)
    flat_tables = tbl_rm.reshape(NLINES, EMBED_DIM)
    xt = x.T.astype(jnp.int32)
    f = jnp.arange(N_FIELDS, dtype=jnp.int32)[:, None]
    offs = (f // 8) * (VOCAB_PAD // 2 * 8) + (f % 8)
    gidx = ((xt >> 1) * 8 + offs).reshape(NW, NWIN, GWIN)
    parr = (xt & 1).reshape(N_FIELDS, BATCH // 8, 8)
    staged = _sc_gather(flat_tables, gidx)
    stg = staged.reshape(N_FIELDS, BATCH // 8, 128)
    w128 = jnp.tile(W.reshape(N_FIELDS, 1, EMBED_DIM), (1, 1, 8))
    out8 = _tc_reduce(stg, parr, w128, b)
    return out8.reshape(BATCH)
